# Initial kernel scaffold; baseline (speedup 1.0000x reference)
import jax
import jax.numpy as jnp
from jax.experimental import pallas as pl


def kernel(x, edge_index, W1, b1, W2, b2):
    raise NotImplementedError("write your pallas kernel here")


# speedup vs baseline: 12.9390x; 12.9390x over previous
import jax
import jax.numpy as jnp
from jax import lax
from jax.experimental import pallas as pl
from jax.experimental.pallas import tpu as pltpu
from jax.experimental.pallas import tpu_sc as plsc

N_NODES = 10000
N_EDGES = 320000
D = 128
NP = 10240
NC, NS = 2, 16
NW = NC * NS
EPT = N_EDGES // NW
B = 80
NCH = EPT // B
RPS = NP // NS
RB = 1024
GRID = NP // RB

_mesh = plsc.VectorSubcoreMesh(
    core_axis_name="c", subcore_axis_name="s", num_cores=NC, num_subcores=NS
)


def _deg_body(dst_hbm, deg_out, idx_v, ones_v, zrow_v, acc):
    cid = lax.axis_index("c")
    sid = lax.axis_index("s")
    wid = sid * NC + cid
    one16 = jnp.full((16,), 1.0, jnp.float32)
    zero16 = jnp.zeros((16,), jnp.float32)
    for i in range(B // 16):
        ones_v[pl.ds(i * 16, 16)] = one16
    for i in range(RPS // 16):
        zrow_v[pl.ds(i * 16, 16)] = zero16
    ro = pl.multiple_of(sid * RPS, 8)
    pltpu.sync_copy(zrow_v, acc.at[pl.ds(ro, RPS)])
    plsc.subcore_barrier()
    base = wid * EPT

    def chunk(j, carry):
        off = pl.multiple_of(base + j * B, 8)
        pltpu.sync_copy(dst_hbm.at[pl.ds(off, B)], idx_v)
        pltpu.sync_copy(ones_v, acc.at[idx_v], add=True)
        return carry

    lax.fori_loop(0, NCH, chunk, 0)
    plsc.subcore_barrier()
    wo = pl.multiple_of(cid * NP + sid * RPS, 8)
    pltpu.sync_copy(acc.at[pl.ds(ro, RPS)], deg_out.at[pl.ds(wo, RPS)])


_deg_call = pl.kernel(
    _deg_body,
    out_type=jax.ShapeDtypeStruct((NC * NP,), jnp.float32),
    mesh=_mesh,
    scratch_types=[
        pltpu.VMEM((B,), jnp.int32),
        pltpu.VMEM((B,), jnp.float32),
        pltpu.VMEM((RPS,), jnp.float32),
        pltpu.VMEM_SHARED((NP,), jnp.float32),
    ],
)


def _agg_body(src_hbm, dst_hbm, g_hbm, out_hbm, srcv, dstv, rows, acc, sem):
    cid = lax.axis_index("c")
    sid = lax.axis_index("s")
    wid = sid * NC + cid
    ro = pl.multiple_of(sid * RPS, 8)
    pltpu.sync_copy(g_hbm.at[pl.ds(ro, RPS)], acc.at[pl.ds(ro, RPS)])
    plsc.subcore_barrier()
    base = wid * EPT

    def chunk(j, carry):
        off = pl.multiple_of(base + j * B, 8)
        pltpu.sync_copy(src_hbm.at[pl.ds(off, B)], srcv)
        pltpu.sync_copy(dst_hbm.at[pl.ds(off, B)], dstv)
        pltpu.async_copy(g_hbm.at[srcv], rows, sem).wait()
        pltpu.sync_copy(rows, acc.at[dstv], add=True)
        return carry

    lax.fori_loop(0, NCH, chunk, 0)
    plsc.subcore_barrier()
    wo = pl.multiple_of(cid * NP + sid * RPS, 8)
    pltpu.sync_copy(acc.at[pl.ds(ro, RPS)], out_hbm.at[pl.ds(wo, RPS)])


_agg_call = pl.kernel(
    _agg_body,
    out_type=jax.ShapeDtypeStruct((NC * NP, D), jnp.float32),
    mesh=_mesh,
    scratch_types=[
        pltpu.VMEM((B,), jnp.int32),
        pltpu.VMEM((B,), jnp.int32),
        pltpu.VMEM((B, D), jnp.float32),
        pltpu.VMEM_SHARED((NP, D), jnp.float32),
        pltpu.SemaphoreType.DMA,
    ],
)


def _tc1_body(x_ref, w_ref, d0_ref, d1_ref, o_ref):
    dinv = lax.rsqrt(d0_ref[0] + d1_ref[0] + 1.0)
    h = jnp.dot(x_ref[...], w_ref[...], preferred_element_type=jnp.float32)
    o_ref[...] = h * dinv


def _tc2_body(p0_ref, p1_ref, g_ref, d0_ref, d1_ref, b_ref, w_ref, o_ref):
    dinv = lax.rsqrt(d0_ref[0] + d1_ref[0] + 1.0)
    pre = (p0_ref[...] + p1_ref[...] - g_ref[...]) * dinv + b_ref[...]
    z = jnp.maximum(pre, 0.0)
    h = jnp.dot(z, w_ref[...], preferred_element_type=jnp.float32)
    o_ref[...] = h * dinv


def _tc3_body(p0_ref, p1_ref, g_ref, d0_ref, d1_ref, b_ref, o_ref):
    dinv = lax.rsqrt(d0_ref[0] + d1_ref[0] + 1.0)
    o_ref[...] = (p0_ref[...] + p1_ref[...] - g_ref[...]) * dinv + b_ref[...]


_row_spec = pl.BlockSpec((RB, D), lambda i: (i, 0))
_row2_spec = pl.BlockSpec((RB, D), lambda i: (i + GRID, 0))
_d0_spec = pl.BlockSpec((1, RB, 1), lambda i: (0, i, 0))
_d1_spec = pl.BlockSpec((1, RB, 1), lambda i: (1, i, 0))
_w_spec = pl.BlockSpec((D, D), lambda i: (0, 0))
_b_spec = pl.BlockSpec((1, D), lambda i: (0, 0))
_out_t = jax.ShapeDtypeStruct((NP, D), jnp.float32)

_tc1 = pl.pallas_call(
    _tc1_body,
    grid=(GRID,),
    in_specs=[_row_spec, _w_spec, _d0_spec, _d1_spec],
    out_specs=_row_spec,
    out_shape=_out_t,
)

_tc2 = pl.pallas_call(
    _tc2_body,
    grid=(GRID,),
    in_specs=[_row_spec, _row2_spec, _row_spec, _d0_spec, _d1_spec, _b_spec, _w_spec],
    out_specs=_row_spec,
    out_shape=_out_t,
)

_tc3 = pl.pallas_call(
    _tc3_body,
    grid=(GRID,),
    in_specs=[_row_spec, _row2_spec, _row_spec, _d0_spec, _d1_spec, _b_spec],
    out_specs=_row_spec,
    out_shape=_out_t,
)


def kernel(x, edge_index, W1, b1, W2, b2):
    src = edge_index[0].astype(jnp.int32)
    dst = edge_index[1].astype(jnp.int32)
    x_pad = jnp.pad(x, ((0, NP - N_NODES), (0, 0)))
    deg3 = _deg_call(dst).reshape(NC, NP, 1)
    g1 = _tc1(x_pad, W1, deg3, deg3)
    p1 = _agg_call(src, dst, g1)
    g2 = _tc2(p1, p1, g1, deg3, deg3, b1.reshape(1, D), W2)
    p2 = _agg_call(src, dst, g2)
    out = _tc3(p2, p2, g2, deg3, deg3, b2.reshape(1, D))
    return out[:N_NODES]
